# Initial kernel scaffold; baseline (speedup 1.0000x reference)
import jax
import jax.numpy as jnp
from jax.experimental import pallas as pl


def kernel(buffer, key):
    raise NotImplementedError("write your pallas kernel here")


# speedup vs baseline: 1.0165x; 1.0165x over previous
import functools

import jax
import jax.numpy as jnp
from jax import lax
from jax.experimental import pallas as pl
from jax.experimental.pallas import tpu as pltpu
from jax.experimental.pallas import tpu_sc as plsc

_BUFFER_SIZE = 100000
_D = 128
_MINVAL = -1.0
_MAXVAL = 1.0
_NUM_CHAINS = 4096
_N_NEW = int(_NUM_CHAINS * 0.05)
_N_OLD = _NUM_CHAINS - _N_NEW

_NW = 32
_ROWS_PER_W = _NUM_CHAINS // _NW


def _sc_body(buf_hbm, idx_hbm, new_hbm, out_hbm, idx_v, rows_v, sem):
    wid = lax.axis_index("s") * 2 + lax.axis_index("c")
    base = wid * _ROWS_PER_W

    @pl.when(wid > 0)
    def _gather():
        pltpu.sync_copy(idx_hbm.at[pl.ds(base, _ROWS_PER_W)], idx_v)
        pltpu.async_copy(buf_hbm.at[idx_v], rows_v, sem).wait()

    @pl.when(wid == 0)
    def _new0():
        pltpu.sync_copy(new_hbm.at[pl.ds(0, _ROWS_PER_W)], rows_v)

    @pl.when(wid == 1)
    def _new1():
        pltpu.sync_copy(
            new_hbm.at[pl.ds(_ROWS_PER_W, _N_NEW - _ROWS_PER_W)],
            rows_v.at[pl.ds(0, _N_NEW - _ROWS_PER_W)],
        )

    pltpu.sync_copy(rows_v, out_hbm.at[pl.ds(base, _ROWS_PER_W)])


_mesh = plsc.VectorSubcoreMesh(core_axis_name="c", subcore_axis_name="s")

_assemble = functools.partial(
    pl.kernel,
    mesh=_mesh,
    out_type=jax.ShapeDtypeStruct((_NUM_CHAINS, _D), jnp.float32),
    scratch_types=[
        pltpu.VMEM((_ROWS_PER_W,), jnp.int32),
        pltpu.VMEM((_ROWS_PER_W, _D), jnp.float32),
        pltpu.SemaphoreType.DMA,
    ],
)(_sc_body)


def kernel(buffer, key):
    key, subkey = jax.random.split(key, 2)
    new_samples = jax.random.uniform(
        subkey, minval=_MINVAL, maxval=_MAXVAL, shape=(_N_NEW, _D))
    key, subkey = jax.random.split(key, 2)
    idx = jax.random.randint(subkey, (_N_OLD,), 0, _BUFFER_SIZE)
    idx_full = jnp.concatenate([jnp.zeros((_N_NEW,), jnp.int32), idx])
    return _assemble(buffer, idx_full, new_samples)


# speedup vs baseline: 1.4862x; 1.4621x over previous
import functools

import jax
import jax.numpy as jnp
from jax import lax
from jax.experimental import pallas as pl
from jax.experimental.pallas import tpu as pltpu
from jax.experimental.pallas import tpu_sc as plsc

_BUFFER_SIZE = 100000
_D = 128
_NUM_CHAINS = 4096
_N_NEW = 204
_N_OLD = _NUM_CHAINS - _N_NEW

_U32 = jnp.uint32


def _u32c(x):
    return _U32(x)


def _rotl(x, r):
    return (x << _u32c(r)) | (x >> _u32c(32 - r))


_ROT0 = (13, 15, 26, 6)
_ROT1 = (17, 29, 16, 24)


def _threefry(k0, k1, x0, x1):
    ks2 = k0 ^ k1 ^ _u32c(0x1BD11BDA)
    ks = (k0, k1, ks2)
    x0 = x0 + k0
    x1 = x1 + k1
    for g in range(5):
        for r in (_ROT0 if g % 2 == 0 else _ROT1):
            x0 = x0 + x1
            x1 = _rotl(x1, r)
            x1 = x1 ^ x0
        x0 = x0 + ks[(g + 1) % 3]
        x1 = x1 + ks[(g + 2) % 3] + _u32c(g + 1)
    return x0, x1


def _uniform_chunk(s10, s11, zeros, lanes_u, flat_base):
    i = flat_base.astype(_U32) + lanes_u
    o0, o1 = _threefry(s10, s11, zeros, i)
    b = ((o0 ^ o1) >> _u32c(9)) | _u32c(0x3F800000)
    f = lax.bitcast_convert_type(b, jnp.float32) - jnp.float32(1.0)
    return jnp.maximum(jnp.float32(-1.0),
                       f * jnp.float32(2.0) + jnp.float32(-1.0))


def _sc_body(buf_hbm, key_hbm, out_hbm, kidx_v, kv, idx_v, new_v, mix_v,
             rows_v, sem):
    wid = lax.axis_index("s") * 2 + lax.axis_index("c")
    zeros = jnp.zeros((16,), _U32)
    lanes = lax.broadcasted_iota(jnp.int32, (16,), 0)
    lanes_u = lanes.astype(_U32)

    kidx_v[pl.ds(0, 16)] = jnp.zeros((16,), jnp.int32)
    kidx_v[pl.ds(16, 16)] = jnp.ones((16,), jnp.int32)
    pltpu.async_copy(key_hbm.at[kidx_v], kv, sem).wait()
    k0s = kv[pl.ds(0, 16)].astype(_U32)
    k1s = kv[pl.ds(16, 16)].astype(_U32)

    ones = jnp.full((16,), _u32c(1))
    a0, a1 = _threefry(k0s, k1s, zeros, zeros)
    s10, s11 = _threefry(k0s, k1s, zeros, ones)
    s20, s21 = _threefry(a0, a1, zeros, ones)
    c20, c21 = _threefry(s20, s21, zeros, ones)

    span = jnp.full((16,), _u32c(_BUFFER_SIZE))

    old_base = jnp.where(wid == 31, 0, 4 + jnp.minimum(wid * 128, 3760))
    for c in range(8):
        j = (old_base + c * 16).astype(_U32) + lanes_u
        o0, o1 = _threefry(c20, c21, zeros, j)
        idx_v[pl.ds(c * 16, 16)] = ((o0 ^ o1) % span).astype(jnp.int32)

    @pl.when(wid < 31)
    def _gather_start():
        pltpu.async_copy(buf_hbm.at[idx_v], rows_v, sem)

    @pl.when(wid == 31)
    def _gather_mixed_start():
        pltpu.async_copy(buf_hbm.at[idx_v.at[pl.ds(0, 4)]],
                         mix_v.at[pl.ds(4, 4)], sem)

    @pl.when(wid < 25)
    def _new_slab():
        start = wid * 8

        def _row(r, carry):
            base = (start + r) * _D
            for c in range(8):
                new_v[r, pl.ds(c * 16, 16)] = _uniform_chunk(
                    s10, s11, zeros, lanes_u, base + c * 16)
            return carry

        lax.fori_loop(0, 8, _row, 0)
        pltpu.sync_copy(new_v,
                        out_hbm.at[pl.ds(pl.multiple_of(start, 8), 8)])

    @pl.when(wid == 31)
    def _mixed_block():
        def _row(r, carry):
            base = (200 + r) * _D
            for c in range(8):
                mix_v[r, pl.ds(c * 16, 16)] = _uniform_chunk(
                    s10, s11, zeros, lanes_u, base + c * 16)
            return carry

        lax.fori_loop(0, 4, _row, 0)

    @pl.when(wid < 31)
    def _old_slab():
        pltpu.make_async_copy(buf_hbm.at[idx_v], rows_v, sem).wait()
        start = 208 + jnp.minimum(wid * 128, 3760)
        pltpu.sync_copy(rows_v,
                        out_hbm.at[pl.ds(pl.multiple_of(start, 8), 128)])

    @pl.when(wid == 31)
    def _mixed_store():
        pltpu.make_async_copy(buf_hbm.at[idx_v.at[pl.ds(0, 4)]],
                              mix_v.at[pl.ds(4, 4)], sem).wait()
        pltpu.sync_copy(mix_v, out_hbm.at[pl.ds(200, 8)])


_mesh = plsc.VectorSubcoreMesh(core_axis_name="c", subcore_axis_name="s")

_sample = functools.partial(
    pl.kernel,
    mesh=_mesh,
    out_type=jax.ShapeDtypeStruct((_NUM_CHAINS, _D), jnp.float32),
    scratch_types=[
        pltpu.VMEM((32,), jnp.int32),
        pltpu.VMEM((32,), jnp.int32),
        pltpu.VMEM((128,), jnp.int32),
        pltpu.VMEM((8, _D), jnp.float32),
        pltpu.VMEM((8, _D), jnp.float32),
        pltpu.VMEM((128, _D), jnp.float32),
        pltpu.SemaphoreType.DMA,
    ],
)(_sc_body)


def kernel(buffer, key):
    kd = lax.bitcast_convert_type(jax.random.key_data(key), jnp.int32)
    return _sample(buffer, kd)


# speedup vs baseline: 1.5253x; 1.0263x over previous
import functools

import jax
import jax.numpy as jnp
from jax import lax
from jax.experimental import pallas as pl
from jax.experimental.pallas import tpu as pltpu
from jax.experimental.pallas import tpu_sc as plsc

_BUFFER_SIZE = 100000
_D = 128
_NUM_CHAINS = 4096
_N_NEW = 204
_N_OLD = _NUM_CHAINS - _N_NEW

_U32 = jnp.uint32


def _u32c(x):
    return _U32(x)


def _rotl(x, r):
    return (x << _u32c(r)) | (x >> _u32c(32 - r))


_ROT0 = (13, 15, 26, 6)
_ROT1 = (17, 29, 16, 24)


def _threefry(k0, k1, x0, x1):
    ks2 = k0 ^ k1 ^ _u32c(0x1BD11BDA)
    ks = (k0, k1, ks2)
    x0 = x0 + k0
    x1 = x1 + k1
    for g in range(5):
        for r in (_ROT0 if g % 2 == 0 else _ROT1):
            x0 = x0 + x1
            x1 = _rotl(x1, r)
            x1 = x1 ^ x0
        x0 = x0 + ks[(g + 1) % 3]
        x1 = x1 + ks[(g + 2) % 3] + _u32c(g + 1)
    return x0, x1


def _uniform_chunk(s10, s11, zeros, lanes_u, flat_base):
    i = flat_base.astype(_U32) + lanes_u
    o0, o1 = _threefry(s10, s11, zeros, i)
    b = ((o0 ^ o1) >> _u32c(9)) | _u32c(0x3F800000)
    f = lax.bitcast_convert_type(b, jnp.float32) - jnp.float32(1.0)
    return jnp.maximum(jnp.float32(-1.0),
                       f * jnp.float32(2.0) + jnp.float32(-1.0))


def _sc_body(buf_hbm, key_hbm, out_hbm, kidx_v, kv, idx_v, new_v, mix_v,
             rows_v, sem, sem_b, sem_c):
    wid = lax.axis_index("s") * 2 + lax.axis_index("c")
    zeros = jnp.zeros((16,), _U32)
    lanes = lax.broadcasted_iota(jnp.int32, (16,), 0)
    lanes_u = lanes.astype(_U32)

    kidx_v[pl.ds(0, 16)] = jnp.zeros((16,), jnp.int32)
    kidx_v[pl.ds(16, 16)] = jnp.ones((16,), jnp.int32)
    pltpu.async_copy(key_hbm.at[kidx_v], kv, sem).wait()
    k0s = kv[pl.ds(0, 16)].astype(_U32)
    k1s = kv[pl.ds(16, 16)].astype(_U32)

    ones = jnp.full((16,), _u32c(1))
    a0, a1 = _threefry(k0s, k1s, zeros, zeros)
    s10, s11 = _threefry(k0s, k1s, zeros, ones)
    s20, s21 = _threefry(a0, a1, zeros, ones)
    c20, c21 = _threefry(s20, s21, zeros, ones)

    span = jnp.full((16,), _u32c(_BUFFER_SIZE))

    old_base = jnp.where(wid == 31, 0, 4 + jnp.minimum(wid * 128, 3760))
    for c in range(4):
        j = (old_base + c * 16).astype(_U32) + lanes_u
        o0, o1 = _threefry(c20, c21, zeros, j)
        idx_v[pl.ds(c * 16, 16)] = ((o0 ^ o1) % span).astype(jnp.int32)

    @pl.when(wid < 31)
    def _gather_start_a():
        pltpu.async_copy(buf_hbm.at[idx_v.at[pl.ds(0, 64)]],
                         rows_v.at[pl.ds(0, 64)], sem)

    @pl.when(wid == 31)
    def _gather_mixed_start():
        pltpu.async_copy(buf_hbm.at[idx_v.at[pl.ds(0, 4)]],
                         mix_v.at[pl.ds(4, 4)], sem)

    for c in range(4, 8):
        j = (old_base + c * 16).astype(_U32) + lanes_u
        o0, o1 = _threefry(c20, c21, zeros, j)
        idx_v[pl.ds(c * 16, 16)] = ((o0 ^ o1) % span).astype(jnp.int32)

    @pl.when(wid < 31)
    def _gather_start_b():
        pltpu.async_copy(buf_hbm.at[idx_v.at[pl.ds(64, 64)]],
                         rows_v.at[pl.ds(64, 64)], sem_b)

    @pl.when(wid < 25)
    def _new_slab():
        start = wid * 8

        def _row(r, carry):
            base = (start + r) * _D
            for c in range(8):
                new_v[r, pl.ds(c * 16, 16)] = _uniform_chunk(
                    s10, s11, zeros, lanes_u, base + c * 16)
            return carry

        lax.fori_loop(0, 8, _row, 0)
        pltpu.sync_copy(new_v,
                        out_hbm.at[pl.ds(pl.multiple_of(start, 8), 8)])

    @pl.when(wid == 31)
    def _mixed_block():
        def _row(r, carry):
            base = (200 + r) * _D
            for c in range(8):
                mix_v[r, pl.ds(c * 16, 16)] = _uniform_chunk(
                    s10, s11, zeros, lanes_u, base + c * 16)
            return carry

        lax.fori_loop(0, 4, _row, 0)

    @pl.when(wid < 31)
    def _old_slab():
        start = 208 + jnp.minimum(wid * 128, 3760)
        start = pl.multiple_of(start, 8)
        pltpu.make_async_copy(buf_hbm.at[idx_v.at[pl.ds(0, 64)]],
                              rows_v.at[pl.ds(0, 64)], sem).wait()
        pltpu.async_copy(rows_v.at[pl.ds(0, 64)],
                         out_hbm.at[pl.ds(start, 64)], sem_c)
        pltpu.make_async_copy(buf_hbm.at[idx_v.at[pl.ds(64, 64)]],
                              rows_v.at[pl.ds(64, 64)], sem_b).wait()
        pltpu.sync_copy(rows_v.at[pl.ds(64, 64)],
                        out_hbm.at[pl.ds(start + 64, 64)])
        pltpu.make_async_copy(rows_v.at[pl.ds(0, 64)],
                              out_hbm.at[pl.ds(start, 64)], sem_c).wait()

    @pl.when(wid == 31)
    def _mixed_store():
        pltpu.make_async_copy(buf_hbm.at[idx_v.at[pl.ds(0, 4)]],
                              mix_v.at[pl.ds(4, 4)], sem).wait()
        pltpu.sync_copy(mix_v, out_hbm.at[pl.ds(200, 8)])


_mesh = plsc.VectorSubcoreMesh(core_axis_name="c", subcore_axis_name="s")

_sample = functools.partial(
    pl.kernel,
    mesh=_mesh,
    out_type=jax.ShapeDtypeStruct((_NUM_CHAINS, _D), jnp.float32),
    scratch_types=[
        pltpu.VMEM((32,), jnp.int32),
        pltpu.VMEM((32,), jnp.int32),
        pltpu.VMEM((128,), jnp.int32),
        pltpu.VMEM((8, _D), jnp.float32),
        pltpu.VMEM((8, _D), jnp.float32),
        pltpu.VMEM((128, _D), jnp.float32),
        pltpu.SemaphoreType.DMA,
        pltpu.SemaphoreType.DMA,
        pltpu.SemaphoreType.DMA,
    ],
)(_sc_body)


def kernel(buffer, key):
    kd = lax.bitcast_convert_type(jax.random.key_data(key), jnp.int32)
    return _sample(buffer, kd)


# speedup vs baseline: 1.6866x; 1.1058x over previous
import functools

import jax
import jax.numpy as jnp
from jax import lax
from jax.experimental import pallas as pl
from jax.experimental.pallas import tpu as pltpu
from jax.experimental.pallas import tpu_sc as plsc

_BUFFER_SIZE = 100000
_D = 128
_NUM_CHAINS = 4096
_N_NEW = 204
_N_OLD = _NUM_CHAINS - _N_NEW

_U32 = jnp.uint32


def _u32c(x):
    return _U32(x)


def _rotl(x, r):
    return (x << _u32c(r)) | (x >> _u32c(32 - r))


_ROT0 = (13, 15, 26, 6)
_ROT1 = (17, 29, 16, 24)


def _threefry(k0, k1, x0, x1):
    ks2 = k0 ^ k1 ^ _u32c(0x1BD11BDA)
    ks = (k0, k1, ks2)
    x0 = x0 + k0
    x1 = x1 + k1
    for g in range(5):
        for r in (_ROT0 if g % 2 == 0 else _ROT1):
            x0 = x0 + x1
            x1 = _rotl(x1, r)
            x1 = x1 ^ x0
        x0 = x0 + ks[(g + 1) % 3]
        x1 = x1 + ks[(g + 2) % 3] + _u32c(g + 1)
    return x0, x1


def _uniform_chunk(s10, s11, zeros, lanes_u, flat_base):
    i = flat_base.astype(_U32) + lanes_u
    o0, o1 = _threefry(s10, s11, zeros, i)
    b = ((o0 ^ o1) >> _u32c(9)) | _u32c(0x3F800000)
    f = lax.bitcast_convert_type(b, jnp.float32) - jnp.float32(1.0)
    return jnp.maximum(jnp.float32(-1.0),
                       f * jnp.float32(2.0) + jnp.float32(-1.0))


def _sc_body(buf_hbm, key_hbm, out_hbm, kidx_v, kv, idx_v, new_v, mix_v,
             rows_v, sem, sem_b, sem_c):
    wid = lax.axis_index("s") * 2 + lax.axis_index("c")
    zeros = jnp.zeros((16,), _U32)
    lanes = lax.broadcasted_iota(jnp.int32, (16,), 0)
    lanes_u = lanes.astype(_U32)

    kidx_v[pl.ds(0, 16)] = jnp.zeros((16,), jnp.int32)
    kidx_v[pl.ds(16, 16)] = jnp.ones((16,), jnp.int32)
    pltpu.async_copy(key_hbm.at[kidx_v], kv, sem).wait()
    k0s = kv[pl.ds(0, 16)].astype(_U32)
    k1s = kv[pl.ds(16, 16)].astype(_U32)

    ones = jnp.full((16,), _u32c(1))
    a0, a1 = _threefry(k0s, k1s, zeros, zeros)
    s10, s11 = _threefry(k0s, k1s, zeros, ones)
    s20, s21 = _threefry(a0, a1, zeros, ones)
    c20, c21 = _threefry(s20, s21, zeros, ones)

    span = jnp.full((16,), _u32c(_BUFFER_SIZE))

    old_base = jnp.where(wid == 31, 0, 4 + jnp.minimum(wid * 128, 3760))

    def _idx_chunk(c, carry):
        j = (old_base + c * 16).astype(_U32) + lanes_u
        o0, o1 = _threefry(c20, c21, zeros, j)
        idx_v[pl.ds(c * 16, 16)] = ((o0 ^ o1) % span).astype(jnp.int32)
        return carry

    lax.fori_loop(0, 8, _idx_chunk, 0)

    @pl.when(wid < 31)
    def _gather_start():
        pltpu.async_copy(buf_hbm.at[idx_v], rows_v, sem)

    @pl.when(wid == 31)
    def _gather_mixed_start():
        pltpu.async_copy(buf_hbm.at[idx_v.at[pl.ds(0, 4)]],
                         mix_v.at[pl.ds(4, 4)], sem)

    @pl.when(wid < 25)
    def _new_slab():
        start = wid * 8

        def _chunk(c, carry):
            r = c // 8
            col = (c % 8) * 16
            new_v[r, pl.ds(col, 16)] = _uniform_chunk(
                s10, s11, zeros, lanes_u, start * _D + c * 16)
            return carry

        lax.fori_loop(0, 64, _chunk, 0)
        pltpu.sync_copy(new_v,
                        out_hbm.at[pl.ds(pl.multiple_of(start, 8), 8)])

    @pl.when(wid == 31)
    def _mixed_block():
        def _chunk(c, carry):
            r = c // 8
            col = (c % 8) * 16
            mix_v[r, pl.ds(col, 16)] = _uniform_chunk(
                s10, s11, zeros, lanes_u, 200 * _D + c * 16)
            return carry

        lax.fori_loop(0, 32, _chunk, 0)

    @pl.when(wid < 31)
    def _old_slab():
        start = 208 + jnp.minimum(wid * 128, 3760)
        pltpu.make_async_copy(buf_hbm.at[idx_v], rows_v, sem).wait()
        pltpu.sync_copy(rows_v,
                        out_hbm.at[pl.ds(pl.multiple_of(start, 8), 128)])

    @pl.when(wid == 31)
    def _mixed_store():
        pltpu.make_async_copy(buf_hbm.at[idx_v.at[pl.ds(0, 4)]],
                              mix_v.at[pl.ds(4, 4)], sem).wait()
        pltpu.sync_copy(mix_v, out_hbm.at[pl.ds(200, 8)])


_mesh = plsc.VectorSubcoreMesh(core_axis_name="c", subcore_axis_name="s")

_sample = functools.partial(
    pl.kernel,
    mesh=_mesh,
    out_type=jax.ShapeDtypeStruct((_NUM_CHAINS, _D), jnp.float32),
    scratch_types=[
        pltpu.VMEM((32,), jnp.int32),
        pltpu.VMEM((32,), jnp.int32),
        pltpu.VMEM((128,), jnp.int32),
        pltpu.VMEM((8, _D), jnp.float32),
        pltpu.VMEM((8, _D), jnp.float32),
        pltpu.VMEM((128, _D), jnp.float32),
        pltpu.SemaphoreType.DMA,
        pltpu.SemaphoreType.DMA,
        pltpu.SemaphoreType.DMA,
    ],
)(_sc_body)


def kernel(buffer, key):
    kd = lax.bitcast_convert_type(jax.random.key_data(key), jnp.int32)
    return _sample(buffer, kd)
